# baseline (speedup 1.0000x reference)
import functools

import jax
import jax.numpy as jnp
from jax import lax
from jax.experimental import pallas as pl
from jax.experimental.pallas import tpu as pltpu
from jax.experimental.pallas import tpu_sc as plsc

E = 8
LANES = 16
NUM_WORKERS = 32


NBUF = 4


def _mm_body(chunk, nchunks, x_hbm, w_ref, o_ref, xbufs, sems):
    def start(i):
        pltpu.make_async_copy(
            x_hbm.at[pl.ds(i * chunk, chunk), :],
            xbufs.at[lax.rem(i, NBUF)],
            sems.at[lax.rem(i, NBUF)],
        ).start()

    for i in range(NBUF - 1):
        start(jnp.int32(i))

    def body(i, carry):
        @pl.when(i + NBUF - 1 < nchunks)
        def _():
            start(i + NBUF - 1)
        rb = lax.rem(i, NBUF)
        pltpu.make_async_copy(
            x_hbm.at[pl.ds(i * chunk, chunk), :],
            xbufs.at[rb],
            sems.at[rb],
        ).wait()
        o_ref[:, pl.ds(i * chunk, chunk)] = lax.dot_general(
            w_ref[...], xbufs[rb],
            dimension_numbers=(((1,), (1,)), ((), ())),
            preferred_element_type=jnp.float32,
        )
        return carry

    lax.fori_loop(0, nchunks, body, jnp.int32(0))


def _logits_t(x, weight, tb, start_blk, nblk):
    t, h = x.shape
    return pl.pallas_call(
        _mm_block_body,
        grid=(nblk,),
        in_specs=[
            pl.BlockSpec((tb, h), lambda i: (start_blk + i, 0)),
            pl.BlockSpec((E, h), lambda i: (0, 0)),
        ],
        out_specs=pl.BlockSpec((E, tb), lambda i: (0, i)),
        out_shape=jax.ShapeDtypeStruct((E, nblk * tb), jnp.float32),
    )(x, weight)


def _mm_block_body(x_ref, w_ref, o_ref):
    o_ref[...] = lax.dot_general(
        w_ref[...], x_ref[...],
        dimension_numbers=(((1,), (1,)), ((), ())),
        preferred_element_type=jnp.float32,
    )


def _route_body(tpw, logits_hbm, i1_hbm, i2_hbm, w1_hbm, w2_hbm,
                lbuf, ibuf1, ibuf2, wbuf1, wbuf2):
    wid = lax.axis_index("s") * 2 + lax.axis_index("c")
    base = wid * tpw
    pltpu.sync_copy(logits_hbm.at[:, pl.ds(base, tpw)], lbuf)

    def body(g, carry):
        off = g * LANES
        vs = [lbuf[e, pl.ds(off, LANES)] for e in range(E)]
        m = vs[0]
        for e in range(1, E):
            m = jnp.maximum(m, vs[e])
        qs = [jnp.exp(v - m) for v in vs]
        s = qs[0]
        for e in range(1, E):
            s = s + qs[e]
        ps = [q / s for q in qs]
        m1 = ps[0]
        for e in range(1, E):
            m1 = jnp.maximum(m1, ps[e])
        i1 = jnp.full((LANES,), E - 1, jnp.int32)
        for e in range(E - 1, -1, -1):
            i1 = jnp.where(ps[e] == m1, jnp.full((LANES,), e, jnp.int32), i1)
        pm = [jnp.where(i1 == jnp.full((LANES,), e, jnp.int32),
                        jnp.full((LANES,), -1.0, jnp.float32), ps[e])
              for e in range(E)]
        m2 = pm[0]
        for e in range(1, E):
            m2 = jnp.maximum(m2, pm[e])
        i2 = jnp.full((LANES,), E - 1, jnp.int32)
        for e in range(E - 1, -1, -1):
            i2 = jnp.where(pm[e] == m2, jnp.full((LANES,), e, jnp.int32), i2)
        ibuf1[pl.ds(off, LANES)] = i1
        ibuf2[pl.ds(off, LANES)] = i2
        wbuf1[pl.ds(off, LANES)] = m1
        wbuf2[pl.ds(off, LANES)] = m2
        return carry

    lax.fori_loop(0, tpw // LANES, body, jnp.int32(0))
    pltpu.sync_copy(ibuf1, i1_hbm.at[pl.ds(base, tpw)])
    pltpu.sync_copy(ibuf2, i2_hbm.at[pl.ds(base, tpw)])
    pltpu.sync_copy(wbuf1, w1_hbm.at[pl.ds(base, tpw)])
    pltpu.sync_copy(wbuf2, w2_hbm.at[pl.ds(base, tpw)])


def _route(logits_t):
    _, t = logits_t.shape
    tpw = t // NUM_WORKERS
    mesh = plsc.VectorSubcoreMesh(core_axis_name="c", subcore_axis_name="s")
    run = pl.kernel(
        functools.partial(_route_body, tpw),
        out_type=(
            jax.ShapeDtypeStruct((t,), jnp.int32),
            jax.ShapeDtypeStruct((t,), jnp.int32),
            jax.ShapeDtypeStruct((t,), jnp.float32),
            jax.ShapeDtypeStruct((t,), jnp.float32),
        ),
        mesh=mesh,
        compiler_params=pltpu.CompilerParams(needs_layout_passes=False),
        scratch_types=[
            pltpu.VMEM((E, tpw), jnp.float32),
            pltpu.VMEM((tpw,), jnp.int32),
            pltpu.VMEM((tpw,), jnp.int32),
            pltpu.VMEM((tpw,), jnp.float32),
            pltpu.VMEM((tpw,), jnp.float32),
        ],
    )
    return run(logits_t)


CHUNKS = 2
TB = 1024


def kernel(hidden_states, weight):
    bsz, seq, h = hidden_states.shape
    t = bsz * seq
    x = hidden_states.reshape(t, h)
    nblk = t // TB // CHUNKS
    planes = []
    for c in range(CHUNKS):
        logits_t = _logits_t(x, weight, TB, c * nblk, nblk)
        planes.append(_route(logits_t))
    i1, i2, w1, w2 = (jnp.concatenate(p) for p in zip(*planes))
    return jnp.stack([i1, i2], axis=1), jnp.stack([w1, w2], axis=1)


# speedup vs baseline: 1.0108x; 1.0108x over previous
import functools

import jax
import jax.numpy as jnp
from jax import lax
from jax.experimental import pallas as pl
from jax.experimental.pallas import tpu as pltpu
from jax.experimental.pallas import tpu_sc as plsc

E = 8
LANES = 16
NUM_WORKERS = 32


NBUF = 4


def _mm_body(chunk, nchunks, x_hbm, w_ref, o_ref, xbufs, sems):
    def start(i):
        pltpu.make_async_copy(
            x_hbm.at[pl.ds(i * chunk, chunk), :],
            xbufs.at[lax.rem(i, NBUF)],
            sems.at[lax.rem(i, NBUF)],
        ).start()

    for i in range(NBUF - 1):
        start(jnp.int32(i))

    def body(i, carry):
        @pl.when(i + NBUF - 1 < nchunks)
        def _():
            start(i + NBUF - 1)
        rb = lax.rem(i, NBUF)
        pltpu.make_async_copy(
            x_hbm.at[pl.ds(i * chunk, chunk), :],
            xbufs.at[rb],
            sems.at[rb],
        ).wait()
        o_ref[:, pl.ds(i * chunk, chunk)] = lax.dot_general(
            w_ref[...], xbufs[rb],
            dimension_numbers=(((1,), (1,)), ((), ())),
            preferred_element_type=jnp.float32,
        )
        return carry

    lax.fori_loop(0, nchunks, body, jnp.int32(0))


def _logits_t(x, weight, tb, start_blk, nblk):
    t, h = x.shape
    return pl.pallas_call(
        _mm_block_body,
        grid=(nblk,),
        in_specs=[
            pl.BlockSpec((tb, h), lambda i: (start_blk + i, 0)),
            pl.BlockSpec((E, h), lambda i: (0, 0)),
        ],
        out_specs=pl.BlockSpec((E, tb), lambda i: (0, i)),
        out_shape=jax.ShapeDtypeStruct((E, nblk * tb), jnp.float32),
    )(x, weight)


def _mm_block_body(x_ref, w_ref, o_ref):
    o_ref[...] = lax.dot_general(
        w_ref[...], x_ref[...],
        dimension_numbers=(((1,), (1,)), ((), ())),
        preferred_element_type=jnp.float32,
    )


def _route_body(tpw, logits_hbm, i1_hbm, i2_hbm, w1_hbm, w2_hbm,
                lbuf, ibuf1, ibuf2, wbuf1, wbuf2):
    wid = lax.axis_index("s") * 2 + lax.axis_index("c")
    base = wid * tpw
    pltpu.sync_copy(logits_hbm.at[:, pl.ds(base, tpw)], lbuf)

    def body(g, carry):
        off = g * LANES
        vs = [lbuf[e, pl.ds(off, LANES)] for e in range(E)]
        m = vs[0]
        for e in range(1, E):
            m = jnp.maximum(m, vs[e])
        qs = [jnp.exp(v - m) for v in vs]
        s = qs[0]
        for e in range(1, E):
            s = s + qs[e]
        m1 = qs[0]
        for e in range(1, E):
            m1 = jnp.maximum(m1, qs[e])
        i1 = jnp.full((LANES,), E - 1, jnp.int32)
        for e in range(E - 1, -1, -1):
            i1 = jnp.where(qs[e] == m1, jnp.full((LANES,), e, jnp.int32), i1)
        pm = [jnp.where(i1 == jnp.full((LANES,), e, jnp.int32),
                        jnp.full((LANES,), -1.0, jnp.float32), qs[e])
              for e in range(E)]
        m2 = pm[0]
        for e in range(1, E):
            m2 = jnp.maximum(m2, pm[e])
        i2 = jnp.full((LANES,), E - 1, jnp.int32)
        for e in range(E - 1, -1, -1):
            i2 = jnp.where(pm[e] == m2, jnp.full((LANES,), e, jnp.int32), i2)
        ibuf1[pl.ds(off, LANES)] = i1
        ibuf2[pl.ds(off, LANES)] = i2
        wbuf1[pl.ds(off, LANES)] = m1 / s
        wbuf2[pl.ds(off, LANES)] = m2 / s
        return carry

    lax.fori_loop(0, tpw // LANES, body, jnp.int32(0))
    pltpu.sync_copy(ibuf1, i1_hbm.at[pl.ds(base, tpw)])
    pltpu.sync_copy(ibuf2, i2_hbm.at[pl.ds(base, tpw)])
    pltpu.sync_copy(wbuf1, w1_hbm.at[pl.ds(base, tpw)])
    pltpu.sync_copy(wbuf2, w2_hbm.at[pl.ds(base, tpw)])


def _route(logits_t):
    _, t = logits_t.shape
    tpw = t // NUM_WORKERS
    mesh = plsc.VectorSubcoreMesh(core_axis_name="c", subcore_axis_name="s")
    run = pl.kernel(
        functools.partial(_route_body, tpw),
        out_type=(
            jax.ShapeDtypeStruct((t,), jnp.int32),
            jax.ShapeDtypeStruct((t,), jnp.int32),
            jax.ShapeDtypeStruct((t,), jnp.float32),
            jax.ShapeDtypeStruct((t,), jnp.float32),
        ),
        mesh=mesh,
        compiler_params=pltpu.CompilerParams(needs_layout_passes=False),
        scratch_types=[
            pltpu.VMEM((E, tpw), jnp.float32),
            pltpu.VMEM((tpw,), jnp.int32),
            pltpu.VMEM((tpw,), jnp.int32),
            pltpu.VMEM((tpw,), jnp.float32),
            pltpu.VMEM((tpw,), jnp.float32),
        ],
    )
    return run(logits_t)


TB = 1024
SPLIT = 28


def kernel(hidden_states, weight):
    bsz, seq, h = hidden_states.shape
    t = bsz * seq
    x = hidden_states.reshape(t, h)
    nblk = t // TB
    planes = []
    for start, n in ((0, SPLIT), (SPLIT, nblk - SPLIT)):
        logits_t = _logits_t(x, weight, TB, start, n)
        planes.append(_route(logits_t))
    i1, i2, w1, w2 = (jnp.concatenate(p) for p in zip(*planes))
    return jnp.stack([i1, i2], axis=1), jnp.stack([w1, w2], axis=1)


# speedup vs baseline: 1.0132x; 1.0024x over previous
import functools

import jax
import jax.numpy as jnp
from jax import lax
from jax.experimental import pallas as pl
from jax.experimental.pallas import tpu as pltpu
from jax.experimental.pallas import tpu_sc as plsc

E = 8
LANES = 16
NUM_WORKERS = 32


def _logits_t(x, weight, tb, start_blk, nblk):
    t, h = x.shape
    return pl.pallas_call(
        _mm_block_body,
        grid=(nblk,),
        in_specs=[
            pl.BlockSpec((tb, h), lambda i: (start_blk + i, 0)),
            pl.BlockSpec((E, h), lambda i: (0, 0)),
        ],
        out_specs=pl.BlockSpec((E, tb), lambda i: (0, i)),
        out_shape=jax.ShapeDtypeStruct((E, nblk * tb), jnp.float32),
    )(x, weight)


def _mm_block_body(x_ref, w_ref, o_ref):
    o_ref[...] = lax.dot_general(
        w_ref[...], x_ref[...],
        dimension_numbers=(((1,), (1,)), ((), ())),
        preferred_element_type=jnp.float32,
    )


def _route_body(tpw, logits_hbm, i1_hbm, i2_hbm, w1_hbm, w2_hbm,
                lbuf, ibuf1, ibuf2, wbuf1, wbuf2):
    wid = lax.axis_index("s") * 2 + lax.axis_index("c")
    base = wid * tpw
    pltpu.sync_copy(logits_hbm.at[:, pl.ds(base, tpw)], lbuf)

    def body(g, carry):
        off = g * LANES
        vs = [lbuf[e, pl.ds(off, LANES)] for e in range(E)]
        m = vs[0]
        for e in range(1, E):
            m = jnp.maximum(m, vs[e])
        qs = [jnp.exp(v - m) for v in vs]
        s = qs[0]
        for e in range(1, E):
            s = s + qs[e]
        m1 = qs[0]
        for e in range(1, E):
            m1 = jnp.maximum(m1, qs[e])
        i1 = jnp.full((LANES,), E - 1, jnp.int32)
        for e in range(E - 1, -1, -1):
            i1 = jnp.where(qs[e] == m1, jnp.full((LANES,), e, jnp.int32), i1)
        pm = [jnp.where(i1 == jnp.full((LANES,), e, jnp.int32),
                        jnp.full((LANES,), -1.0, jnp.float32), qs[e])
              for e in range(E)]
        m2 = pm[0]
        for e in range(1, E):
            m2 = jnp.maximum(m2, pm[e])
        i2 = jnp.full((LANES,), E - 1, jnp.int32)
        for e in range(E - 1, -1, -1):
            i2 = jnp.where(pm[e] == m2, jnp.full((LANES,), e, jnp.int32), i2)
        ibuf1[pl.ds(off, LANES)] = i1
        ibuf2[pl.ds(off, LANES)] = i2
        wbuf1[pl.ds(off, LANES)] = m1 / s
        wbuf2[pl.ds(off, LANES)] = m2 / s
        return carry

    lax.fori_loop(0, tpw // LANES, body, jnp.int32(0))
    pltpu.sync_copy(ibuf1, i1_hbm.at[pl.ds(base, tpw)])
    pltpu.sync_copy(ibuf2, i2_hbm.at[pl.ds(base, tpw)])
    pltpu.sync_copy(wbuf1, w1_hbm.at[pl.ds(base, tpw)])
    pltpu.sync_copy(wbuf2, w2_hbm.at[pl.ds(base, tpw)])


def _route(logits_t):
    _, t = logits_t.shape
    tpw = t // NUM_WORKERS
    mesh = plsc.VectorSubcoreMesh(core_axis_name="c", subcore_axis_name="s")
    run = pl.kernel(
        functools.partial(_route_body, tpw),
        out_type=(
            jax.ShapeDtypeStruct((t,), jnp.int32),
            jax.ShapeDtypeStruct((t,), jnp.int32),
            jax.ShapeDtypeStruct((t,), jnp.float32),
            jax.ShapeDtypeStruct((t,), jnp.float32),
        ),
        mesh=mesh,
        compiler_params=pltpu.CompilerParams(needs_layout_passes=False),
        scratch_types=[
            pltpu.VMEM((E, tpw), jnp.float32),
            pltpu.VMEM((tpw,), jnp.int32),
            pltpu.VMEM((tpw,), jnp.int32),
            pltpu.VMEM((tpw,), jnp.float32),
            pltpu.VMEM((tpw,), jnp.float32),
        ],
    )
    return run(logits_t)


TB = 1024
SPLIT = 28


def kernel(hidden_states, weight):
    bsz, seq, h = hidden_states.shape
    t = bsz * seq
    x = hidden_states.reshape(t, h)
    nblk = t // TB
    planes = []
    for start, n in ((0, SPLIT), (SPLIT, nblk - SPLIT)):
        logits_t = _logits_t(x, weight, TB, start, n)
        planes.append(_route(logits_t))
    i1, i2, w1, w2 = (jnp.concatenate(p) for p in zip(*planes))
    return jnp.stack([i1, i2], axis=1), jnp.stack([w1, w2], axis=1)
